# baseline (device time: 21944 ns/iter reference)
import jax
import jax.numpy as jnp
from jax import lax
from jax.experimental import pallas as pl
from jax.experimental.pallas import tpu as pltpu

N_DEV = 8


def kernel(x, router_W, route_idx, expert_W):
    n_tok, d = x.shape
    n_exp = router_W.shape[1]
    e_local, _, h = expert_W.shape
    rows = n_tok // N_DEV

    def body(x_ref, rw_ref, idx_ref, ew_ref, out_ref,
             partial_ref, acc_ref, recv_ref, send_sems, recv_sems):
        my = lax.axis_index("i")
        left = lax.rem(my + N_DEV - 1, N_DEV)
        right = lax.rem(my + 1, N_DEV)

        barrier_sem = pltpu.get_barrier_semaphore()
        for nbr in (left, right):
            pl.semaphore_signal(
                barrier_sem, inc=1,
                device_id=(nbr,), device_id_type=pl.DeviceIdType.MESH,
            )
        pl.semaphore_wait(barrier_sem, 2)

        xv = x_ref[:, :]
        scores = jnp.dot(xv, rw_ref[:, :],
                         preferred_element_type=jnp.float32)
        m = jnp.max(scores, axis=1, keepdims=True)
        p = jnp.exp(scores - m)
        p = p / jnp.sum(p, axis=1, keepdims=True)

        e0 = idx_ref[:, 0:1]
        e1 = idx_ref[:, 1:2]
        iota = lax.broadcasted_iota(jnp.int32, (n_tok, n_exp), 1)
        g0 = jnp.sum(jnp.where(iota == e0, p, 0.0), axis=1, keepdims=True)
        g1 = jnp.sum(jnp.where(iota == e1, p, 0.0), axis=1, keepdims=True)
        denom = g0 + g1

        partial = jnp.zeros((n_tok, h), dtype=jnp.float32)
        for j in range(e_local):
            e = my * e_local + j
            ge = jnp.sum(jnp.where(iota == e, p, 0.0), axis=1, keepdims=True)
            sel = (e0 == e) | (e1 == e)
            w = jnp.where(sel, ge / denom, 0.0)
            partial = partial + w * jnp.dot(
                xv, ew_ref[j], preferred_element_type=jnp.float32)
        partial_ref[:, :] = partial

        first = lax.rem(my + N_DEV - 1, N_DEV)
        acc_ref[:, :] = partial_ref[pl.ds(first * rows, rows), :]
        for s in range(N_DEV - 1):
            rdma = pltpu.make_async_remote_copy(
                src_ref=acc_ref,
                dst_ref=recv_ref.at[s],
                send_sem=send_sems.at[s],
                recv_sem=recv_sems.at[s],
                device_id=(right,),
                device_id_type=pl.DeviceIdType.MESH,
            )
            rdma.start()
            rdma.wait()
            c = lax.rem(my + 2 * N_DEV - 2 - s, N_DEV)
            reduced = recv_ref[s] + partial_ref[pl.ds(c * rows, rows), :]
            if s < N_DEV - 2:
                acc_ref[:, :] = reduced
            else:
                out_ref[:, :] = reduced

    return pl.pallas_call(
        body,
        out_shape=jax.ShapeDtypeStruct((rows, h), jnp.float32),
        in_specs=[
            pl.BlockSpec(memory_space=pltpu.VMEM),
            pl.BlockSpec(memory_space=pltpu.VMEM),
            pl.BlockSpec(memory_space=pltpu.VMEM),
            pl.BlockSpec(memory_space=pltpu.VMEM),
        ],
        out_specs=pl.BlockSpec(memory_space=pltpu.VMEM),
        scratch_shapes=[
            pltpu.VMEM((n_tok, h), jnp.float32),
            pltpu.VMEM((rows, h), jnp.float32),
            pltpu.VMEM((N_DEV - 1, rows, h), jnp.float32),
            pltpu.SemaphoreType.DMA((N_DEV - 1,)),
            pltpu.SemaphoreType.DMA((N_DEV - 1,)),
        ],
        compiler_params=pltpu.CompilerParams(collective_id=0),
    )(x, router_W, route_idx, expert_W)


# device time: 11109 ns/iter; 1.9753x vs baseline; 1.9753x over previous
import jax
import jax.numpy as jnp
from jax import lax
from jax.experimental import pallas as pl
from jax.experimental.pallas import tpu as pltpu

N_DEV = 8


def kernel(x, router_W, route_idx, expert_W):
    n_tok, d = x.shape
    n_exp = router_W.shape[1]
    e_local, _, h = expert_W.shape
    rows = n_tok // N_DEV

    def body(x_ref, rw_ref, idx_ref, ew_ref, out_ref,
             partial_ref, recv_ref, send_sems, recv_sems):
        my = lax.axis_index("i")

        barrier_sem = pltpu.get_barrier_semaphore()
        for k in range(1, N_DEV):
            pl.semaphore_signal(
                barrier_sem, inc=1,
                device_id=(lax.rem(my + k, N_DEV),),
                device_id_type=pl.DeviceIdType.MESH,
            )
        pl.semaphore_wait(barrier_sem, N_DEV - 1)

        xv = x_ref[:, :]
        scores = jnp.dot(xv, rw_ref[:, :],
                         preferred_element_type=jnp.float32)
        m = jnp.max(scores, axis=1, keepdims=True)
        p = jnp.exp(scores - m)
        p = p / jnp.sum(p, axis=1, keepdims=True)

        e0 = idx_ref[:, 0:1]
        e1 = idx_ref[:, 1:2]
        iota = lax.broadcasted_iota(jnp.int32, (n_tok, n_exp), 1)
        g0 = jnp.sum(jnp.where(iota == e0, p, 0.0), axis=1, keepdims=True)
        g1 = jnp.sum(jnp.where(iota == e1, p, 0.0), axis=1, keepdims=True)
        denom = g0 + g1

        partial = jnp.zeros((n_tok, h), dtype=jnp.float32)
        for j in range(e_local):
            e = my * e_local + j
            ge = jnp.sum(jnp.where(iota == e, p, 0.0), axis=1, keepdims=True)
            sel = (e0 == e) | (e1 == e)
            w = jnp.where(sel, ge / denom, 0.0)
            partial = partial + w * jnp.dot(
                xv, ew_ref[j], preferred_element_type=jnp.float32)
        partial_ref[:, :] = partial

        rdmas = []
        for k in range(1, N_DEV):
            t = lax.rem(my + k, N_DEV)
            rdma = pltpu.make_async_remote_copy(
                src_ref=partial_ref.at[pl.ds(t * rows, rows), :],
                dst_ref=recv_ref.at[k - 1],
                send_sem=send_sems.at[k - 1],
                recv_sem=recv_sems.at[k - 1],
                device_id=(t,),
                device_id_type=pl.DeviceIdType.MESH,
            )
            rdma.start()
            rdmas.append(rdma)

        acc = partial_ref[pl.ds(my * rows, rows), :]
        for k in range(1, N_DEV):
            rdmas[k - 1].wait_recv()
            acc = acc + recv_ref[k - 1]
        out_ref[:, :] = acc

        for k in range(1, N_DEV):
            rdmas[k - 1].wait_send()

    return pl.pallas_call(
        body,
        out_shape=jax.ShapeDtypeStruct((rows, h), jnp.float32),
        in_specs=[
            pl.BlockSpec(memory_space=pltpu.VMEM),
            pl.BlockSpec(memory_space=pltpu.VMEM),
            pl.BlockSpec(memory_space=pltpu.VMEM),
            pl.BlockSpec(memory_space=pltpu.VMEM),
        ],
        out_specs=pl.BlockSpec(memory_space=pltpu.VMEM),
        scratch_shapes=[
            pltpu.VMEM((n_tok, h), jnp.float32),
            pltpu.VMEM((N_DEV - 1, rows, h), jnp.float32),
            pltpu.SemaphoreType.DMA((N_DEV - 1,)),
            pltpu.SemaphoreType.DMA((N_DEV - 1,)),
        ],
        compiler_params=pltpu.CompilerParams(collective_id=0),
    )(x, router_W, route_idx, expert_W)
